# initial kernel scaffold (unmeasured)
import functools

import jax
import jax.numpy as jnp
from jax import lax
from jax.experimental import pallas as pl
from jax.experimental.pallas import tpu as pltpu

N_DEV = 8
B, SQ, D = 2, 128, 512
HQ_LOCAL, DH = 8, 64
M = B * SQ


def kernel(x, Wq, Wo, K_ext, V_ext):
    def body(x_ref, wq_ref, wo_ref, k_ref, v_ref, out_ref,
             comm_ref, send_ref, send_sems, recv_sems):
        my_pos = lax.axis_index("i")

        barrier_sem = pltpu.get_barrier_semaphore()
        for k in range(1, N_DEV):
            peer = lax.rem(my_pos + k, N_DEV)
            pl.semaphore_signal(
                barrier_sem, inc=1,
                device_id=(peer,), device_id_type=pl.DeviceIdType.MESH,
            )
        pl.semaphore_wait(barrier_sem, N_DEV - 1)

        x2d = x_ref[:].reshape(M, D)
        q = jnp.dot(x2d, wq_ref[:], preferred_element_type=jnp.float32)
        q = q.reshape(B, SQ, HQ_LOCAL, DH)

        h0 = my_pos * HQ_LOCAL
        kh = k_ref[:, :, pl.ds(h0, HQ_LOCAL), :]
        vh = v_ref[:, :, pl.ds(h0, HQ_LOCAL), :]

        s = jnp.einsum("bihd,bjhd->bhij", q, kh,
                       preferred_element_type=jnp.float32) * 0.125
        m = jnp.max(s, axis=-1, keepdims=True)
        p = jnp.exp(s - m)
        l = jnp.sum(p, axis=-1, keepdims=True)
        attn = jnp.einsum("bhij,bjhd->bihd", p / l, vh,
                          preferred_element_type=jnp.float32)
        attn2d = attn.reshape(M, HQ_LOCAL * DH)
        partial = jnp.dot(attn2d, wo_ref[:], preferred_element_type=jnp.float32)

        send_ref[:] = partial
        comm_ref[my_pos] = partial

        sends = []
        for k in range(1, N_DEV):
            peer = lax.rem(my_pos + k, N_DEV)
            rdma = pltpu.make_async_remote_copy(
                src_ref=send_ref,
                dst_ref=comm_ref.at[my_pos],
                send_sem=send_sems.at[k],
                recv_sem=recv_sems.at[my_pos],
                device_id=(peer,),
                device_id_type=pl.DeviceIdType.MESH,
            )
            rdma.start()
            sends.append(rdma)

        for k in range(1, N_DEV):
            src_peer = lax.rem(my_pos + k, N_DEV)
            recv = pltpu.make_async_remote_copy(
                src_ref=send_ref,
                dst_ref=comm_ref.at[src_peer],
                send_sem=send_sems.at[0],
                recv_sem=recv_sems.at[src_peer],
                device_id=(src_peer,),
                device_id_type=pl.DeviceIdType.MESH,
            )
            recv.wait_recv()

        total = jnp.sum(comm_ref[:], axis=0)
        out_ref[:] = total.reshape(B, SQ, D)

        for rdma in sends:
            rdma.wait_send()

    return pl.pallas_call(
        body,
        out_shape=jax.ShapeDtypeStruct((B, SQ, D), jnp.float32),
        in_specs=[pl.BlockSpec(memory_space=pltpu.VMEM)] * 5,
        out_specs=pl.BlockSpec(memory_space=pltpu.VMEM),
        scratch_shapes=[
            pltpu.VMEM((N_DEV, M, D), jnp.float32),
            pltpu.VMEM((M, D), jnp.float32),
            pltpu.SemaphoreType.DMA((N_DEV,)),
            pltpu.SemaphoreType.DMA((N_DEV,)),
        ],
        compiler_params=pltpu.CompilerParams(collective_id=0),
    )(x, Wq, Wo, K_ext, V_ext)


# baseline (device time: 61422 ns/iter reference)
import functools

import jax
import jax.numpy as jnp
from jax import lax
from jax.experimental import pallas as pl
from jax.experimental.pallas import tpu as pltpu

N_DEV = 8
B, SQ, D = 2, 128, 512
HQ_LOCAL, DH = 8, 64
M = B * SQ


def kernel(x, Wq, Wo, K_ext, V_ext):
    def body(x_ref, wq_ref, wo_ref, k_ref, v_ref, out_ref,
             comm_ref, send_ref, send_sems, recv_sems):
        my_pos = lax.axis_index("i")

        barrier_sem = pltpu.get_barrier_semaphore()
        for k in range(1, N_DEV):
            peer = lax.rem(my_pos + k, N_DEV)
            pl.semaphore_signal(
                barrier_sem, inc=1,
                device_id=(peer,), device_id_type=pl.DeviceIdType.MESH,
            )
        pl.semaphore_wait(barrier_sem, N_DEV - 1)

        x2d = x_ref[:].reshape(M, D)
        q = jnp.dot(x2d, wq_ref[:], preferred_element_type=jnp.float32)
        q = q.reshape(B, SQ, HQ_LOCAL, DH)

        h0 = my_pos * HQ_LOCAL
        kh = k_ref[:, :, pl.ds(h0, HQ_LOCAL), :]
        vh = v_ref[:, :, pl.ds(h0, HQ_LOCAL), :]

        rows = []
        for b in range(B):
            cols = []
            for h in range(HQ_LOCAL):
                qh = q[b, :, h, :]
                khh = kh[b, :, h, :]
                vhh = vh[b, :, h, :]
                s = lax.dot_general(
                    qh, khh, (((1,), (1,)), ((), ())),
                    preferred_element_type=jnp.float32,
                ) * 0.125
                mx = jnp.max(s, axis=-1, keepdims=True)
                p = jnp.exp(s - mx)
                l = jnp.sum(p, axis=-1, keepdims=True)
                cols.append(jnp.dot(p / l, vhh,
                                    preferred_element_type=jnp.float32))
            rows.append(jnp.concatenate(cols, axis=1))
        attn2d = jnp.concatenate(rows, axis=0)
        partial = jnp.dot(attn2d, wo_ref[:], preferred_element_type=jnp.float32)

        send_ref[:] = partial
        comm_ref[my_pos] = partial

        sends = []
        for k in range(1, N_DEV):
            peer = lax.rem(my_pos + k, N_DEV)
            rdma = pltpu.make_async_remote_copy(
                src_ref=send_ref,
                dst_ref=comm_ref.at[my_pos],
                send_sem=send_sems.at[k],
                recv_sem=recv_sems.at[my_pos],
                device_id=(peer,),
                device_id_type=pl.DeviceIdType.MESH,
            )
            rdma.start()
            sends.append(rdma)

        for k in range(1, N_DEV):
            src_peer = lax.rem(my_pos + k, N_DEV)
            recv = pltpu.make_async_remote_copy(
                src_ref=send_ref,
                dst_ref=comm_ref.at[src_peer],
                send_sem=send_sems.at[0],
                recv_sem=recv_sems.at[src_peer],
                device_id=(src_peer,),
                device_id_type=pl.DeviceIdType.MESH,
            )
            recv.wait_recv()

        total = jnp.sum(comm_ref[:], axis=0)
        out_ref[:] = total.reshape(B, SQ, D)

        for rdma in sends:
            rdma.wait_send()

    return pl.pallas_call(
        body,
        out_shape=jax.ShapeDtypeStruct((B, SQ, D), jnp.float32),
        in_specs=[pl.BlockSpec(memory_space=pltpu.VMEM)] * 5,
        out_specs=pl.BlockSpec(memory_space=pltpu.VMEM),
        scratch_shapes=[
            pltpu.VMEM((N_DEV, M, D), jnp.float32),
            pltpu.VMEM((M, D), jnp.float32),
            pltpu.SemaphoreType.DMA((N_DEV,)),
            pltpu.SemaphoreType.DMA((N_DEV,)),
        ],
        compiler_params=pltpu.CompilerParams(collective_id=0),
    )(x, Wq, Wo, K_ext, V_ext)


# device time: 38616 ns/iter; 1.5906x vs baseline; 1.5906x over previous
import functools

import jax
import jax.numpy as jnp
from jax import lax
from jax.experimental import pallas as pl
from jax.experimental.pallas import tpu as pltpu

N_DEV = 8
B, SQ, D = 2, 128, 512
HQ_LOCAL, DH = 8, 64
M = B * SQ
CHUNK = M // N_DEV


def kernel(x, Wq, Wo, K_ext, V_ext):
    def body(x_ref, wq_ref, wo_ref, k_ref, v_ref, out_ref,
             rs_ref, ag_ref, send_ref,
             rs_send_sems, rs_recv_sems, ag_send_sems, ag_recv_sems):
        my_pos = lax.axis_index("i")

        barrier_sem = pltpu.get_barrier_semaphore()
        for k in range(1, N_DEV):
            peer = lax.rem(my_pos + k, N_DEV)
            pl.semaphore_signal(
                barrier_sem, inc=1,
                device_id=(peer,), device_id_type=pl.DeviceIdType.MESH,
            )
        pl.semaphore_wait(barrier_sem, N_DEV - 1)

        x2d = x_ref[:].reshape(M, D)
        q = jnp.dot(x2d, wq_ref[:], preferred_element_type=jnp.float32)
        q = q.reshape(B, SQ, HQ_LOCAL, DH)

        h0 = my_pos * HQ_LOCAL
        kh = k_ref[:, :, pl.ds(h0, HQ_LOCAL), :]
        vh = v_ref[:, :, pl.ds(h0, HQ_LOCAL), :]

        rows = []
        for b in range(B):
            cols = []
            for h in range(HQ_LOCAL):
                qh = q[b, :, h, :]
                khh = kh[b, :, h, :]
                vhh = vh[b, :, h, :]
                s = lax.dot_general(
                    qh, khh, (((1,), (1,)), ((), ())),
                    preferred_element_type=jnp.float32,
                ) * 0.125
                mx = jnp.max(s, axis=-1, keepdims=True)
                p = jnp.exp(s - mx)
                l = jnp.sum(p, axis=-1, keepdims=True)
                cols.append(jnp.dot(p / l, vhh,
                                    preferred_element_type=jnp.float32))
            rows.append(jnp.concatenate(cols, axis=1))
        attn2d = jnp.concatenate(rows, axis=0)
        partial = jnp.dot(attn2d, wo_ref[:], preferred_element_type=jnp.float32)

        send_ref[:] = partial
        rs_ref[my_pos] = send_ref[pl.ds(my_pos * CHUNK, CHUNK), :]

        rs_sends = []
        for k in range(1, N_DEV):
            peer = lax.rem(my_pos + k, N_DEV)
            rdma = pltpu.make_async_remote_copy(
                src_ref=send_ref.at[pl.ds(peer * CHUNK, CHUNK), :],
                dst_ref=rs_ref.at[my_pos],
                send_sem=rs_send_sems.at[k],
                recv_sem=rs_recv_sems.at[my_pos],
                device_id=(peer,),
                device_id_type=pl.DeviceIdType.MESH,
            )
            rdma.start()
            rs_sends.append(rdma)

        for k in range(1, N_DEV):
            src_peer = lax.rem(my_pos + k, N_DEV)
            recv = pltpu.make_async_remote_copy(
                src_ref=send_ref.at[pl.ds(0, CHUNK), :],
                dst_ref=rs_ref.at[src_peer],
                send_sem=rs_send_sems.at[0],
                recv_sem=rs_recv_sems.at[src_peer],
                device_id=(src_peer,),
                device_id_type=pl.DeviceIdType.MESH,
            )
            recv.wait_recv()

        ag_ref[my_pos] = jnp.sum(rs_ref[:], axis=0)

        ag_sends = []
        for k in range(1, N_DEV):
            peer = lax.rem(my_pos + k, N_DEV)
            rdma = pltpu.make_async_remote_copy(
                src_ref=ag_ref.at[my_pos],
                dst_ref=ag_ref.at[my_pos],
                send_sem=ag_send_sems.at[k],
                recv_sem=ag_recv_sems.at[my_pos],
                device_id=(peer,),
                device_id_type=pl.DeviceIdType.MESH,
            )
            rdma.start()
            ag_sends.append(rdma)

        for k in range(1, N_DEV):
            src_peer = lax.rem(my_pos + k, N_DEV)
            recv = pltpu.make_async_remote_copy(
                src_ref=ag_ref.at[src_peer],
                dst_ref=ag_ref.at[src_peer],
                send_sem=ag_send_sems.at[0],
                recv_sem=ag_recv_sems.at[src_peer],
                device_id=(src_peer,),
                device_id_type=pl.DeviceIdType.MESH,
            )
            recv.wait_recv()

        out_ref[:] = ag_ref[:].reshape(B, SQ, D)

        for rdma in rs_sends + ag_sends:
            rdma.wait_send()

    return pl.pallas_call(
        body,
        out_shape=jax.ShapeDtypeStruct((B, SQ, D), jnp.float32),
        in_specs=[pl.BlockSpec(memory_space=pltpu.VMEM)] * 5,
        out_specs=pl.BlockSpec(memory_space=pltpu.VMEM),
        scratch_shapes=[
            pltpu.VMEM((N_DEV, CHUNK, D), jnp.float32),
            pltpu.VMEM((N_DEV, CHUNK, D), jnp.float32),
            pltpu.VMEM((M, D), jnp.float32),
            pltpu.SemaphoreType.DMA((N_DEV,)),
            pltpu.SemaphoreType.DMA((N_DEV,)),
            pltpu.SemaphoreType.DMA((N_DEV,)),
            pltpu.SemaphoreType.DMA((N_DEV,)),
        ],
        compiler_params=pltpu.CompilerParams(collective_id=0),
    )(x, Wq, Wo, K_ext, V_ext)
